# Initial kernel scaffold; baseline (speedup 1.0000x reference)
import jax
import jax.numpy as jnp
from jax.experimental import pallas as pl


def kernel(X, edge_index, W, b):
    raise NotImplementedError("write your pallas kernel here")


# speedup vs baseline: 8.1031x; 8.1031x over previous
import functools
import math

import jax
import jax.numpy as jnp
from jax import lax
from jax.experimental import pallas as pl
from jax.experimental.pallas import tpu as pltpu
from jax.experimental.pallas import tpu_sc as plsc

_NC = 2
_NS = 16
_L = 16
_NW = _NC * _NS
_NHEADS = 4


def _pick_blocking(n: int, d: int) -> tuple[int, int]:
    budget_words = 1_600_000
    best = None
    for bn in range(512, 7, -8):
        if n % bn:
            continue
        q = math.lcm(_NS * 8, bn)
        np_ = ((n + q - 1) // q) * q
        if np_ * d <= budget_words:
            best = (bn, np_)
            break
    if best is None:
        raise ValueError(f"no valid TC blocking for n={n}, d={d}")
    return best


def _chunk_size(ew: int) -> int:
    for cs in range(128, 0, -8):
        if ew % cs == 0:
            return cs
    raise ValueError(f"no valid chunk size for {ew} edges per worker")


def _zero_chunk(rps: int, d: int) -> int:
    best = 1
    for zc in range(1, rps + 1):
        if rps % zc == 0 and zc * d // _L <= 256:
            best = zc
    return best


def _wb_chunk(rps: int, d: int) -> int:
    best = 8
    for wc in range(8, rps + 1, 8):
        if rps % wc == 0 and wc * d * 4 <= 128 * 1024:
            best = wc
    return best


def _mesh():
    return plsc.VectorSubcoreMesh(
        core_axis_name="c", subcore_axis_name="s",
        num_cores=_NC, num_subcores=_NS)


def _make_counts(n: int, np_: int, e: int, d: int):
    ew = e // _NW
    assert ew * _NW == e
    cs = _chunk_size(ew)
    nch = ew // cs
    rps = np_ // _NS
    zc = _zero_chunk(rps, d)
    wc = _wb_chunk(rps, d)

    @functools.partial(
        pl.kernel,
        out_type=jax.ShapeDtypeStruct((_NC * np_, d), jnp.float32),
        mesh=_mesh(),
        scratch_types=[
            pltpu.VMEM((cs,), jnp.int32),
            pltpu.VMEM((cs, d), jnp.float32),
            pltpu.VMEM((zc, d), jnp.float32),
            pltpu.VMEM((wc, d), jnp.float32),
            pltpu.VMEM_SHARED((np_, d), jnp.float32),
        ],
    )
    def counts(eidx_hbm, out_hbm, idx_v, ones_v, zbuf, wbuf, acc):
        c = lax.axis_index("c")
        s = lax.axis_index("s")
        wid = s * _NC + c

        ones16 = jnp.ones((_L,), jnp.float32)
        zeros16 = jnp.zeros((_L,), jnp.float32)
        for r in range(cs):
            ones_v[r, pl.ds(0, _L)] = ones16
            for k in range(1, d // _L):
                ones_v[r, pl.ds(k * _L, _L)] = zeros16
        for r in range(zc):
            for k in range(d // _L):
                zbuf[r, pl.ds(k * _L, _L)] = zeros16

        base_r = s * rps
        for k in range(rps // zc):
            pltpu.sync_copy(zbuf, acc.at[pl.ds(base_r + k * zc, zc)])
        plsc.subcore_barrier()

        base_e = wid * ew

        def step(i, carry):
            off = base_e + i * cs
            pltpu.sync_copy(eidx_hbm.at[pl.ds(off, cs)], idx_v)
            pltpu.sync_copy(ones_v, acc.at[idx_v], add=True)
            return carry

        lax.fori_loop(0, nch, step, 0)
        plsc.subcore_barrier()

        for k in range(rps // wc):
            r0 = base_r + k * wc
            pltpu.sync_copy(acc.at[pl.ds(r0, wc)], wbuf)
            pltpu.sync_copy(wbuf, out_hbm.at[pl.ds(c * np_ + r0, wc)])

    return counts


def _make_rowsum(n: int, np_: int, e: int, d: int):
    ew = e // _NW
    assert ew * _NW == e
    cs = _chunk_size(ew)
    nch = ew // cs
    rps = np_ // _NS
    zc = _zero_chunk(rps, d)
    wc = _wb_chunk(rps, d)

    @functools.partial(
        pl.kernel,
        out_type=jax.ShapeDtypeStruct((_NC * np_, d), jnp.float32),
        mesh=_mesh(),
        scratch_types=[
            pltpu.VMEM((cs,), jnp.int32),
            pltpu.VMEM((cs,), jnp.int32),
            pltpu.VMEM((cs, d), jnp.float32),
            pltpu.VMEM((zc, d), jnp.float32),
            pltpu.VMEM((wc, d), jnp.float32),
            pltpu.VMEM_SHARED((np_, d), jnp.float32),
            pltpu.SemaphoreType.DMA,
        ],
    )
    def rowsum(tab_hbm, gidx_hbm, sidx_hbm, out_hbm, idx_g, idx_a, rows,
               zbuf, wbuf, acc, sem):
        c = lax.axis_index("c")
        s = lax.axis_index("s")
        wid = s * _NC + c

        zeros16 = jnp.zeros((_L,), jnp.float32)
        for r in range(zc):
            for k in range(d // _L):
                zbuf[r, pl.ds(k * _L, _L)] = zeros16

        base_r = s * rps
        for k in range(rps // zc):
            pltpu.sync_copy(zbuf, acc.at[pl.ds(base_r + k * zc, zc)])
        plsc.subcore_barrier()

        base_e = wid * ew

        def step(i, carry):
            off = base_e + i * cs
            pltpu.sync_copy(gidx_hbm.at[pl.ds(off, cs)], idx_g)
            pltpu.sync_copy(sidx_hbm.at[pl.ds(off, cs)], idx_a)
            pltpu.async_copy(tab_hbm.at[idx_g], rows, sem).wait()
            pltpu.sync_copy(rows, acc.at[idx_a], add=True)
            return carry

        lax.fori_loop(0, nch, step, 0)
        plsc.subcore_barrier()

        for k in range(rps // wc):
            r0 = base_r + k * wc
            pltpu.sync_copy(acc.at[pl.ds(r0, wc)], wbuf)
            pltpu.sync_copy(wbuf, out_hbm.at[pl.ds(c * np_ + r0, wc)])

    return rowsum


def _count_extract(cd, nh):
    rid = lax.broadcasted_iota(jnp.int32, (nh, 1), 0)
    p = jnp.where(rid < _L, 1.0 / _L, 0.0).astype(jnp.float32)
    return jnp.dot(cd, p, preferred_element_type=jnp.float32)


def _tc_scale_matmul(X, W, CD, np_, bn):
    n, dx = X.shape
    nh = W.shape[1]
    nb = np_ // bn

    def body(x_ref, w_ref, c0_ref, c1_ref, o_ref):
        deg = jnp.maximum(_count_extract(c0_ref[...] + c1_ref[...], nh), 1.0)
        h = jnp.dot(x_ref[...], w_ref[...], preferred_element_type=jnp.float32)
        o_ref[...] = h * lax.rsqrt(deg)

    return pl.pallas_call(
        body,
        grid=(n // bn,),
        in_specs=[
            pl.BlockSpec((bn, dx), lambda i: (i, 0)),
            pl.BlockSpec((dx, nh), lambda i: (0, 0)),
            pl.BlockSpec((bn, nh), lambda i: (i, 0)),
            pl.BlockSpec((bn, nh), lambda i: (i + nb, 0)),
        ],
        out_specs=pl.BlockSpec((bn, nh), lambda i: (i, 0)),
        out_shape=jax.ShapeDtypeStruct((n, nh), jnp.float32),
    )(X, W, CD, CD)


def _tc_elu_headmean(n, T, CD, b2, np_, bn):
    nh = T.shape[1]
    nhid = nh // _NHEADS
    nb = np_ // bn

    def body(t0_ref, t1_ref, c0_ref, c1_ref, b_ref, y2_ref, xc_ref, xsq_ref):
        deg = jnp.maximum(_count_extract(c0_ref[...] + c1_ref[...], nh), 1.0)
        agg = (t0_ref[...] + t1_ref[...]) * lax.rsqrt(deg) + b_ref[...]
        el = jnp.where(agg > 0.0, agg, jnp.exp(jnp.minimum(agg, 0.0)) - 1.0)
        rid = lax.broadcasted_iota(jnp.int32, (nh, nhid), 0) // _NHEADS
        cid = lax.broadcasted_iota(jnp.int32, (nh, nhid), 1)
        m = jnp.where(rid == cid, 1.0 / _NHEADS, 0.0).astype(jnp.float32)
        xc = jnp.dot(el, m, preferred_element_type=jnp.float32)
        xsq = xc * xc
        ones_blk = jnp.ones((xc.shape[0], _L), jnp.float32)
        pad = jnp.zeros((xc.shape[0], nh - 2 * nhid - _L), jnp.float32)
        y2_ref[...] = jnp.concatenate([xc, xsq, ones_blk, pad], axis=1)
        xc_ref[...] = xc
        xsq_ref[...] = xsq

    return pl.pallas_call(
        body,
        grid=(n // bn,),
        in_specs=[
            pl.BlockSpec((bn, nh), lambda i: (i, 0)),
            pl.BlockSpec((bn, nh), lambda i: (i + nb, 0)),
            pl.BlockSpec((bn, nh), lambda i: (i, 0)),
            pl.BlockSpec((bn, nh), lambda i: (i + nb, 0)),
            pl.BlockSpec((1, nh), lambda i: (0, 0)),
        ],
        out_specs=[
            pl.BlockSpec((bn, nh), lambda i: (i, 0)),
            pl.BlockSpec((bn, nhid), lambda i: (i, 0)),
            pl.BlockSpec((bn, nhid), lambda i: (i, 0)),
        ],
        out_shape=[
            jax.ShapeDtypeStruct((n, nh), jnp.float32),
            jax.ShapeDtypeStruct((n, nhid), jnp.float32),
            jax.ShapeDtypeStruct((n, nhid), jnp.float32),
        ],
    )(T, T, CD, CD, b2)


def _tc_finalize(S, Xc, Xsq, np_, bn):
    n, nhid = Xc.shape
    nh = S.shape[1]
    nb = np_ // bn

    def body(s0_ref, s1_ref, xc_ref, xsq_ref, o_ref):
        sv = s0_ref[...] + s1_ref[...]
        rid = lax.broadcasted_iota(jnp.int32, (nh, nhid), 0)
        cid = lax.broadcasted_iota(jnp.int32, (nh, nhid), 1)
        p_a = jnp.where(rid == cid, 1.0, 0.0).astype(jnp.float32)
        p_b = jnp.where(rid == cid + nhid, 1.0, 0.0).astype(jnp.float32)
        rid1 = lax.broadcasted_iota(jnp.int32, (nh, 1), 0)
        p_c = jnp.where((rid1 >= 2 * nhid) & (rid1 < 2 * nhid + _L),
                        1.0 / _L, 0.0).astype(jnp.float32)
        asum = jnp.dot(sv, p_a, preferred_element_type=jnp.float32)
        bsum = jnp.dot(sv, p_b, preferred_element_type=jnp.float32)
        craw = jnp.dot(sv, p_c, preferred_element_type=jnp.float32)
        c1v = jnp.maximum(craw, 1.0)
        s = craw * xsq_ref[...] - 2.0 * xc_ref[...] * asum + bsum
        o_ref[...] = jnp.tanh(s / c1v)

    return pl.pallas_call(
        body,
        grid=(n // bn,),
        in_specs=[
            pl.BlockSpec((bn, nh), lambda i: (i, 0)),
            pl.BlockSpec((bn, nh), lambda i: (i + nb, 0)),
            pl.BlockSpec((bn, nhid), lambda i: (i, 0)),
            pl.BlockSpec((bn, nhid), lambda i: (i, 0)),
        ],
        out_specs=pl.BlockSpec((bn, nhid), lambda i: (i, 0)),
        out_shape=jax.ShapeDtypeStruct((n, nhid), jnp.float32),
    )(S, S, Xc, Xsq)


def kernel(X, edge_index, W, b):
    n, _ = X.shape
    e = edge_index.shape[1]
    nh = W.shape[1]
    nhid = nh // _NHEADS
    bn, np_ = _pick_blocking(n, W.shape[1])

    src = edge_index[0]
    dst = edge_index[1]

    cd = _make_counts(n, np_, e, nh)(dst)
    hn = _tc_scale_matmul(X, W, cd, np_, bn)
    rs = _make_rowsum(n, np_, e, nh)
    T = rs(hn, src, dst)
    Y2, Xc, Xsq = _tc_elu_headmean(n, T, cd, b.reshape(1, nh), np_, bn)
    S = rs(Y2, dst, src)
    return _tc_finalize(S, Xc, Xsq, np_, bn)


# speedup vs baseline: 9.0166x; 1.1127x over previous
import functools
import math

import jax
import jax.numpy as jnp
from jax import lax
from jax.experimental import pallas as pl
from jax.experimental.pallas import tpu as pltpu
from jax.experimental.pallas import tpu_sc as plsc

_NC = 2
_NS = 16
_L = 16
_NW = _NC * _NS
_NHEADS = 4


def _pick_blocking(n: int, d: int) -> tuple[int, int]:
    budget_words = 1_600_000
    best = None
    for bn in range(512, 7, -8):
        if n % bn:
            continue
        q = math.lcm(_NS * 8, bn)
        np_ = ((n + q - 1) // q) * q
        if np_ * d <= budget_words:
            best = (bn, np_)
            break
    if best is None:
        raise ValueError(f"no valid TC blocking for n={n}, d={d}")
    return best


def _chunk_size(ew: int) -> int:
    for cs in range(128, 0, -8):
        if ew % cs == 0:
            return cs
    raise ValueError(f"no valid chunk size for {ew} edges per worker")


def _zero_chunk(rps: int, d: int) -> int:
    best = 1
    for zc in range(1, rps + 1):
        if rps % zc == 0 and zc * d // _L <= 256:
            best = zc
    return best


def _wb_chunk(rps: int, d: int) -> int:
    best = 8
    for wc in range(8, rps + 1, 8):
        if rps % wc == 0 and wc * d * 4 <= 128 * 1024:
            best = wc
    return best


def _mesh():
    return plsc.VectorSubcoreMesh(
        core_axis_name="c", subcore_axis_name="s",
        num_cores=_NC, num_subcores=_NS)


def _make_counts(n: int, np_: int, e: int, d: int):
    ew = e // _NW
    assert ew * _NW == e
    cs = _chunk_size(ew)
    nch = ew // cs
    rps = np_ // _NS
    zc = _zero_chunk(rps, d)
    wc = _wb_chunk(rps, d)

    @functools.partial(
        pl.kernel,
        out_type=jax.ShapeDtypeStruct((_NC * np_, d), jnp.float32),
        mesh=_mesh(),
        scratch_types=[
            pltpu.VMEM((cs,), jnp.int32),
            pltpu.VMEM((cs, d), jnp.float32),
            pltpu.VMEM((zc, d), jnp.float32),
            pltpu.VMEM((wc, d), jnp.float32),
            pltpu.VMEM_SHARED((np_, d), jnp.float32),
        ],
    )
    def counts(eidx_hbm, out_hbm, idx_v, ones_v, zbuf, wbuf, acc):
        c = lax.axis_index("c")
        s = lax.axis_index("s")
        wid = s * _NC + c

        ones16 = jnp.ones((_L,), jnp.float32)
        zeros16 = jnp.zeros((_L,), jnp.float32)
        for r in range(cs):
            ones_v[r, pl.ds(0, _L)] = ones16
            for k in range(1, d // _L):
                ones_v[r, pl.ds(k * _L, _L)] = zeros16
        for r in range(zc):
            for k in range(d // _L):
                zbuf[r, pl.ds(k * _L, _L)] = zeros16

        base_r = s * rps
        for k in range(rps // zc):
            pltpu.sync_copy(zbuf, acc.at[pl.ds(base_r + k * zc, zc)])
        plsc.subcore_barrier()

        base_e = wid * ew

        def step(i, carry):
            off = base_e + i * cs
            pltpu.sync_copy(eidx_hbm.at[pl.ds(off, cs)], idx_v)
            pltpu.sync_copy(ones_v, acc.at[idx_v], add=True)
            return carry

        lax.fori_loop(0, nch, step, 0)
        plsc.subcore_barrier()

        for k in range(rps // wc):
            r0 = base_r + k * wc
            pltpu.sync_copy(acc.at[pl.ds(r0, wc)], wbuf)
            pltpu.sync_copy(wbuf, out_hbm.at[pl.ds(c * np_ + r0, wc)])

    return counts


def _make_rowsum(n: int, np_: int, e: int, d: int):
    ew = e // _NW
    assert ew * _NW == e
    cs = _chunk_size(ew)
    nch = ew // cs
    rps = np_ // _NS
    zc = _zero_chunk(rps, d)
    wc = _wb_chunk(rps, d)

    @functools.partial(
        pl.kernel,
        out_type=jax.ShapeDtypeStruct((_NC * np_, d), jnp.float32),
        mesh=_mesh(),
        scratch_types=[
            pltpu.VMEM((cs,), jnp.int32),
            pltpu.VMEM((cs,), jnp.int32),
            pltpu.VMEM((cs,), jnp.int32),
            pltpu.VMEM((cs,), jnp.int32),
            pltpu.VMEM((cs, d), jnp.float32),
            pltpu.VMEM((cs, d), jnp.float32),
            pltpu.VMEM((zc, d), jnp.float32),
            pltpu.VMEM((wc, d), jnp.float32),
            pltpu.VMEM_SHARED((np_, d), jnp.float32),
            pltpu.SemaphoreType.DMA,
            pltpu.SemaphoreType.DMA,
            pltpu.SemaphoreType.DMA,
            pltpu.SemaphoreType.DMA,
        ],
    )
    def rowsum(tab_hbm, gidx_hbm, sidx_hbm, out_hbm, ig0, ia0, ig1, ia1,
               rw0, rw1, zbuf, wbuf, acc, sg0, sg1, ss0, ss1):
        c = lax.axis_index("c")
        s = lax.axis_index("s")
        wid = s * _NC + c
        bufs = ((ig0, ia0, rw0, sg0, ss0), (ig1, ia1, rw1, sg1, ss1))

        zeros16 = jnp.zeros((_L,), jnp.float32)
        for r in range(zc):
            for k in range(d // _L):
                zbuf[r, pl.ds(k * _L, _L)] = zeros16

        base_r = s * rps
        for k in range(rps // zc):
            pltpu.sync_copy(zbuf, acc.at[pl.ds(base_r + k * zc, zc)])
        plsc.subcore_barrier()

        base_e = wid * ew

        def fire(ci, p):
            ig, ia, rw, sg, _ = bufs[p]
            off = base_e + ci * cs
            pltpu.sync_copy(gidx_hbm.at[pl.ds(off, cs)], ig)
            pltpu.sync_copy(sidx_hbm.at[pl.ds(off, cs)], ia)
            pltpu.async_copy(tab_hbm.at[ig], rw, sg)

        def half(ci, p, may_drain, may_fire):
            ig, ia, rw, sg, ss = bufs[p]
            _, iaq, rwq, _, ssq = bufs[1 - p]
            pltpu.make_async_copy(tab_hbm.at[ig], rw, sg).wait()
            if may_drain:
                @pl.when(ci >= 1)
                def _():
                    pltpu.make_async_copy(rwq, acc.at[iaq], ssq).wait()
            if may_fire:
                @pl.when(ci + 1 < nch)
                def _():
                    fire(ci + 1, 1 - p)
            pltpu.async_copy(rw, acc.at[ia], ss, add=True)

        fire(0, 0)

        def pair(g, carry):
            half(2 * g, 0, True, True)
            half(2 * g + 1, 1, True, True)
            return carry

        lax.fori_loop(0, nch // 2, pair, 0)
        if nch % 2:
            ci = nch - 1
            half(ci, ci % 2, nch >= 2, False)
        pf = (nch - 1) % 2
        _, iaf, rwf, _, ssf = bufs[pf]
        pltpu.make_async_copy(rwf, acc.at[iaf], ssf).wait()
        plsc.subcore_barrier()

        for k in range(rps // wc):
            r0 = base_r + k * wc
            pltpu.sync_copy(acc.at[pl.ds(r0, wc)], wbuf)
            pltpu.sync_copy(wbuf, out_hbm.at[pl.ds(c * np_ + r0, wc)])

    return rowsum


def _count_extract(cd, nh):
    rid = lax.broadcasted_iota(jnp.int32, (nh, 1), 0)
    p = jnp.where(rid < _L, 1.0 / _L, 0.0).astype(jnp.float32)
    return jnp.dot(cd, p, preferred_element_type=jnp.float32)


def _tc_scale_matmul(X, W, CD, np_, bn):
    n, dx = X.shape
    nh = W.shape[1]
    nb = np_ // bn

    def body(x_ref, w_ref, c0_ref, c1_ref, o_ref):
        deg = jnp.maximum(_count_extract(c0_ref[...] + c1_ref[...], nh), 1.0)
        h = jnp.dot(x_ref[...], w_ref[...], preferred_element_type=jnp.float32)
        o_ref[...] = h * lax.rsqrt(deg)

    return pl.pallas_call(
        body,
        grid=(n // bn,),
        in_specs=[
            pl.BlockSpec((bn, dx), lambda i: (i, 0)),
            pl.BlockSpec((dx, nh), lambda i: (0, 0)),
            pl.BlockSpec((bn, nh), lambda i: (i, 0)),
            pl.BlockSpec((bn, nh), lambda i: (i + nb, 0)),
        ],
        out_specs=pl.BlockSpec((bn, nh), lambda i: (i, 0)),
        out_shape=jax.ShapeDtypeStruct((n, nh), jnp.float32),
    )(X, W, CD, CD)


def _tc_elu_headmean(n, T, CD, b2, np_, bn):
    nh = T.shape[1]
    nhid = nh // _NHEADS
    nb = np_ // bn

    def body(t0_ref, t1_ref, c0_ref, c1_ref, b_ref, y2_ref, xc_ref, xsq_ref):
        deg = jnp.maximum(_count_extract(c0_ref[...] + c1_ref[...], nh), 1.0)
        agg = (t0_ref[...] + t1_ref[...]) * lax.rsqrt(deg) + b_ref[...]
        el = jnp.where(agg > 0.0, agg, jnp.exp(jnp.minimum(agg, 0.0)) - 1.0)
        rid = lax.broadcasted_iota(jnp.int32, (nh, nhid), 0) // _NHEADS
        cid = lax.broadcasted_iota(jnp.int32, (nh, nhid), 1)
        m = jnp.where(rid == cid, 1.0 / _NHEADS, 0.0).astype(jnp.float32)
        xc = jnp.dot(el, m, preferred_element_type=jnp.float32)
        xsq = xc * xc
        ones_blk = jnp.ones((xc.shape[0], _L), jnp.float32)
        pad = jnp.zeros((xc.shape[0], nh - 2 * nhid - _L), jnp.float32)
        y2_ref[...] = jnp.concatenate([xc, xsq, ones_blk, pad], axis=1)
        xc_ref[...] = xc
        xsq_ref[...] = xsq

    return pl.pallas_call(
        body,
        grid=(n // bn,),
        in_specs=[
            pl.BlockSpec((bn, nh), lambda i: (i, 0)),
            pl.BlockSpec((bn, nh), lambda i: (i + nb, 0)),
            pl.BlockSpec((bn, nh), lambda i: (i, 0)),
            pl.BlockSpec((bn, nh), lambda i: (i + nb, 0)),
            pl.BlockSpec((1, nh), lambda i: (0, 0)),
        ],
        out_specs=[
            pl.BlockSpec((bn, nh), lambda i: (i, 0)),
            pl.BlockSpec((bn, nhid), lambda i: (i, 0)),
            pl.BlockSpec((bn, nhid), lambda i: (i, 0)),
        ],
        out_shape=[
            jax.ShapeDtypeStruct((n, nh), jnp.float32),
            jax.ShapeDtypeStruct((n, nhid), jnp.float32),
            jax.ShapeDtypeStruct((n, nhid), jnp.float32),
        ],
    )(T, T, CD, CD, b2)


def _tc_finalize(S, Xc, Xsq, np_, bn):
    n, nhid = Xc.shape
    nh = S.shape[1]
    nb = np_ // bn

    def body(s0_ref, s1_ref, xc_ref, xsq_ref, o_ref):
        sv = s0_ref[...] + s1_ref[...]
        rid = lax.broadcasted_iota(jnp.int32, (nh, nhid), 0)
        cid = lax.broadcasted_iota(jnp.int32, (nh, nhid), 1)
        p_a = jnp.where(rid == cid, 1.0, 0.0).astype(jnp.float32)
        p_b = jnp.where(rid == cid + nhid, 1.0, 0.0).astype(jnp.float32)
        rid1 = lax.broadcasted_iota(jnp.int32, (nh, 1), 0)
        p_c = jnp.where((rid1 >= 2 * nhid) & (rid1 < 2 * nhid + _L),
                        1.0 / _L, 0.0).astype(jnp.float32)
        asum = jnp.dot(sv, p_a, preferred_element_type=jnp.float32)
        bsum = jnp.dot(sv, p_b, preferred_element_type=jnp.float32)
        craw = jnp.dot(sv, p_c, preferred_element_type=jnp.float32)
        c1v = jnp.maximum(craw, 1.0)
        s = craw * xsq_ref[...] - 2.0 * xc_ref[...] * asum + bsum
        o_ref[...] = jnp.tanh(s / c1v)

    return pl.pallas_call(
        body,
        grid=(n // bn,),
        in_specs=[
            pl.BlockSpec((bn, nh), lambda i: (i, 0)),
            pl.BlockSpec((bn, nh), lambda i: (i + nb, 0)),
            pl.BlockSpec((bn, nhid), lambda i: (i, 0)),
            pl.BlockSpec((bn, nhid), lambda i: (i, 0)),
        ],
        out_specs=pl.BlockSpec((bn, nhid), lambda i: (i, 0)),
        out_shape=jax.ShapeDtypeStruct((n, nhid), jnp.float32),
    )(S, S, Xc, Xsq)


def kernel(X, edge_index, W, b):
    n, _ = X.shape
    e = edge_index.shape[1]
    nh = W.shape[1]
    nhid = nh // _NHEADS
    bn, np_ = _pick_blocking(n, W.shape[1])

    src = edge_index[0]
    dst = edge_index[1]

    cd = _make_counts(n, np_, e, nh)(dst)
    hn = _tc_scale_matmul(X, W, cd, np_, bn)
    rs = _make_rowsum(n, np_, e, nh)
    T = rs(hn, src, dst)
    Y2, Xc, Xsq = _tc_elu_headmean(n, T, cd, b.reshape(1, nh), np_, bn)
    S = rs(Y2, dst, src)
    return _tc_finalize(S, Xc, Xsq, np_, bn)


# speedup vs baseline: 14.1440x; 1.5687x over previous
import functools
import math

import jax
import jax.numpy as jnp
from jax import lax
from jax.experimental import pallas as pl
from jax.experimental.pallas import tpu as pltpu
from jax.experimental.pallas import tpu_sc as plsc

_NC = 2
_NS = 16
_L = 16
_NW = _NC * _NS
_NHEADS = 4


def _pick_blocking(n: int, d: int) -> tuple[int, int]:
    budget_words = 1_600_000
    best = None
    for bn in range(512, 7, -8):
        if n % bn:
            continue
        q = math.lcm(_NS * 8, bn)
        np_ = ((n + q - 1) // q) * q
        if np_ * d <= budget_words:
            best = (bn, np_)
            break
    if best is None:
        raise ValueError(f"no valid TC blocking for n={n}, d={d}")
    return best


def _chunking(ew: int) -> tuple[int, int]:
    for cs in range(128, 0, -8):
        if ew % cs == 0:
            return cs, ew // cs
    raise ValueError(f"no valid chunking for {ew} edges per worker")


def _stage_chunk(rps: int, cs: int) -> int:
    for q in range(min(rps, cs) // 8 * 8, 0, -8):
        if rps % q == 0:
            return q
    raise ValueError((rps, cs))


def _zero_chunk(rps: int, d: int) -> int:
    best = 1
    for zc in range(1, rps + 1):
        if rps % zc == 0 and zc * d // _L <= 256:
            best = zc
    return best


def _wb_chunk(rps: int, d: int) -> int:
    best = 8
    for wc in range(8, rps + 1, 8):
        if rps % wc == 0 and wc * d * 4 <= 128 * 1024:
            best = wc
    return best


def _mesh():
    return plsc.VectorSubcoreMesh(
        core_axis_name="c", subcore_axis_name="s",
        num_cores=_NC, num_subcores=_NS)


def _make_counts(n: int, np_: int, e: int, d: int):
    ew = e // _NW
    assert ew * _NW == e
    cs, nch = _chunking(ew)
    assert nch >= 4
    rps = np_ // _NS
    q = _stage_chunk(rps, cs)

    @functools.partial(
        pl.kernel,
        out_type=jax.ShapeDtypeStruct((_NC * np_, d), jnp.float32),
        mesh=_mesh(),
        scratch_types=[
            pltpu.VMEM((cs,), jnp.int32),
            pltpu.VMEM((cs,), jnp.int32),
            pltpu.VMEM((cs, d), jnp.float32),
            pltpu.VMEM((cs, d), jnp.float32),
            pltpu.VMEM_SHARED((np_, d), jnp.float32),
            pltpu.SemaphoreType.DMA,
            pltpu.SemaphoreType.DMA,
        ],
    )
    def counts(eidx_hbm, out_hbm, ia0, ia1, ones_v, zwbuf, acc, ss0, ss1):
        c = lax.axis_index("c")
        s = lax.axis_index("s")
        wid = s * _NC + c
        sems = (ss0, ss1)
        iabufs = (ia0, ia1)
        base_e = wid * ew

        ones16 = jnp.ones((_L,), jnp.float32)
        zeros16 = jnp.zeros((_L,), jnp.float32)
        for r in range(cs):
            ones_v[r, pl.ds(0, _L)] = ones16
            for k in range(1, d // _L):
                ones_v[r, pl.ds(k * _L, _L)] = zeros16
            for k in range(d // _L):
                zwbuf[r, pl.ds(k * _L, _L)] = zeros16

        base_r = s * rps
        for k in range(rps // q):
            pltpu.sync_copy(zwbuf.at[pl.ds(0, q)],
                            acc.at[pl.ds(base_r + k * q, q)])
        plsc.subcore_barrier()

        def chalf(ci, p, guard):
            def drain():
                pltpu.make_async_copy(
                    ones_v, acc.at[iabufs[p]], sems[p]).wait()

            if guard:
                pl.when(ci >= 2)(drain)
            elif ci >= 2:
                drain()
            pltpu.sync_copy(eidx_hbm.at[pl.ds(base_e + ci * cs, cs)],
                            iabufs[p])
            pltpu.async_copy(ones_v, acc.at[iabufs[p]], sems[p], add=True)

        def pair(g, carry):
            chalf(2 * g, 0, True)
            chalf(2 * g + 1, 1, True)
            return carry

        lax.fori_loop(0, nch // 2, pair, 0)
        if nch % 2:
            chalf(nch - 1, (nch - 1) % 2, False)
        pltpu.make_async_copy(ones_v, acc.at[ia0], sems[0]).wait()
        pltpu.make_async_copy(ones_v, acc.at[ia1], sems[1]).wait()
        plsc.subcore_barrier()

        for k in range(rps // q):
            r0 = base_r + k * q
            pltpu.sync_copy(acc.at[pl.ds(r0, q)], zwbuf.at[pl.ds(0, q)])
            pltpu.sync_copy(zwbuf.at[pl.ds(0, q)],
                            out_hbm.at[pl.ds(c * np_ + r0, q)])

    return counts


def _make_rowsum(n: int, np_: int, e: int, d: int):
    ew = e // _NW
    assert ew * _NW == e
    cs, nch = _chunking(ew)
    rps = np_ // _NS
    q = _stage_chunk(rps, cs)

    assert nch >= 6

    @functools.partial(
        pl.kernel,
        out_type=jax.ShapeDtypeStruct((_NC * np_, d), jnp.float32),
        mesh=_mesh(),
        scratch_types=[
            pltpu.VMEM((cs,), jnp.int32),
            pltpu.VMEM((cs,), jnp.int32),
            pltpu.VMEM((cs,), jnp.int32),
            pltpu.VMEM((cs,), jnp.int32),
            pltpu.VMEM((cs,), jnp.int32),
            pltpu.VMEM((cs,), jnp.int32),
            pltpu.VMEM((cs, d), jnp.float32),
            pltpu.VMEM((cs, d), jnp.float32),
            pltpu.VMEM((cs, d), jnp.float32),
            pltpu.VMEM_SHARED((np_, d), jnp.float32),
            pltpu.SemaphoreType.DMA,
            pltpu.SemaphoreType.DMA,
            pltpu.SemaphoreType.DMA,
            pltpu.SemaphoreType.DMA,
            pltpu.SemaphoreType.DMA,
            pltpu.SemaphoreType.DMA,
            pltpu.SemaphoreType.DMA,
            pltpu.SemaphoreType.DMA,
            pltpu.SemaphoreType.DMA,
        ],
    )
    def rowsum(tab_hbm, gidx_hbm, sidx_hbm, out_hbm,
               ig0, ia0, ig1, ia1, ig2, ia2, rw0, rw1, rw2, acc,
               si0, si1, si2, sg0, sg1, sg2, ss0, ss1, ss2):
        c = lax.axis_index("c")
        s = lax.axis_index("s")
        wid = s * _NC + c
        bufs = ((ig0, ia0, rw0, si0, sg0, ss0),
                (ig1, ia1, rw1, si1, sg1, ss1),
                (ig2, ia2, rw2, si2, sg2, ss2))
        base_e = wid * ew

        zeros16 = jnp.zeros((_L,), jnp.float32)
        for r in range(cs):
            for k in range(d // _L):
                rw0[r, pl.ds(k * _L, _L)] = zeros16

        base_r = s * rps
        for k in range(rps // q):
            pltpu.sync_copy(rw0.at[pl.ds(0, q)],
                            acc.at[pl.ds(base_r + k * q, q)])
        plsc.subcore_barrier()

        def idx_fire(ci, j):
            ig, ia, _, si, _, _ = bufs[j]
            off = base_e + ci * cs
            pltpu.async_copy(gidx_hbm.at[pl.ds(off, cs)], ig, si)
            pltpu.async_copy(sidx_hbm.at[pl.ds(off, cs)], ia, si)

        def idx_wait(ci, j):
            ig, ia, _, si, _, _ = bufs[j]
            off = base_e + ci * cs
            pltpu.make_async_copy(gidx_hbm.at[pl.ds(off, cs)], ig, si).wait()
            pltpu.make_async_copy(sidx_hbm.at[pl.ds(off, cs)], ia, si).wait()

        def gather_fire(j):
            ig, _, rw, _, sg, _ = bufs[j]
            pltpu.async_copy(tab_hbm.at[ig], rw, sg)

        def step(ci, j, guard):
            j1 = (j + 1) % 3
            j2 = (j + 2) % 3
            _, ia, rw, _, sg, ss = bufs[j]
            _, iap, rwp, _, _, ssp = bufs[j2]

            def when(pred, f):
                if guard:
                    pl.when(pred)(f)
                elif pred:
                    f()

            when(ci >= 1, lambda: pltpu.make_async_copy(
                rwp, acc.at[iap], ssp).wait())
            when(ci + 2 < nch, lambda: idx_fire(ci + 2, j2))

            def g1():
                idx_wait(ci + 1, j1)
                gather_fire(j1)

            when(ci + 1 < nch, g1)
            pltpu.make_async_copy(tab_hbm.at[bufs[j][0]], rw, sg).wait()
            pltpu.async_copy(rw, acc.at[ia], ss, add=True)

        idx_fire(0, 0)
        idx_fire(1, 1)
        idx_wait(0, 0)
        gather_fire(0)

        def triple(g, carry):
            ci = 3 * g
            step(ci, 0, True)
            step(ci + 1, 1, True)
            step(ci + 2, 2, True)
            return carry

        lax.fori_loop(0, nch // 3, triple, 0)
        for ci in range(nch // 3 * 3, nch):
            step(ci, ci % 3, False)
        jf = (nch - 1) % 3
        _, iaf, rwf, _, _, ssf = bufs[jf]
        pltpu.make_async_copy(rwf, acc.at[iaf], ssf).wait()
        plsc.subcore_barrier()

        for k in range(rps // q):
            r0 = base_r + k * q
            pltpu.sync_copy(acc.at[pl.ds(r0, q)], rw0.at[pl.ds(0, q)])
            pltpu.sync_copy(rw0.at[pl.ds(0, q)],
                            out_hbm.at[pl.ds(c * np_ + r0, q)])

    return rowsum


def _count_extract(cd, nh):
    rid = lax.broadcasted_iota(jnp.int32, (nh, 1), 0)
    p = jnp.where(rid < _L, 1.0 / _L, 0.0).astype(jnp.float32)
    return jnp.dot(cd, p, preferred_element_type=jnp.float32)


def _tc_scale_matmul(X, W, CD, np_, bn):
    n, dx = X.shape
    nh = W.shape[1]
    nb = np_ // bn

    dc = CD.shape[1]

    def body(x_ref, w_ref, c0_ref, c1_ref, o_ref):
        deg = jnp.maximum(_count_extract(c0_ref[...] + c1_ref[...], dc), 1.0)
        h = jnp.dot(x_ref[...], w_ref[...], preferred_element_type=jnp.float32)
        o_ref[...] = h * lax.rsqrt(deg)

    return pl.pallas_call(
        body,
        grid=(n // bn,),
        in_specs=[
            pl.BlockSpec((bn, dx), lambda i: (i, 0)),
            pl.BlockSpec((dx, nh), lambda i: (0, 0)),
            pl.BlockSpec((bn, dc), lambda i: (i, 0)),
            pl.BlockSpec((bn, dc), lambda i: (i + nb, 0)),
        ],
        out_specs=pl.BlockSpec((bn, nh), lambda i: (i, 0)),
        out_shape=jax.ShapeDtypeStruct((n, nh), jnp.float32),
    )(X, W, CD, CD)


def _tc_elu_headmean(n, T, CD, b2, np_, bn):
    nh = T.shape[1]
    nhid = nh // _NHEADS
    nb = np_ // bn

    dc = CD.shape[1]

    def body(t0_ref, t1_ref, c0_ref, c1_ref, b_ref, y2_ref, xc_ref, xsq_ref):
        deg = jnp.maximum(_count_extract(c0_ref[...] + c1_ref[...], dc), 1.0)
        agg = (t0_ref[...] + t1_ref[...]) * lax.rsqrt(deg) + b_ref[...]
        el = jnp.where(agg > 0.0, agg, jnp.exp(jnp.minimum(agg, 0.0)) - 1.0)
        rid = lax.broadcasted_iota(jnp.int32, (nh, nhid), 0) // _NHEADS
        cid = lax.broadcasted_iota(jnp.int32, (nh, nhid), 1)
        m = jnp.where(rid == cid, 1.0 / _NHEADS, 0.0).astype(jnp.float32)
        xc = jnp.dot(el, m, preferred_element_type=jnp.float32)
        xsq = xc * xc
        ones_blk = jnp.ones((xc.shape[0], _L), jnp.float32)
        pad = jnp.zeros((xc.shape[0], nh - 2 * nhid - _L), jnp.float32)
        y2_ref[...] = jnp.concatenate([xc, xsq, ones_blk, pad], axis=1)
        xc_ref[...] = xc
        xsq_ref[...] = xsq

    return pl.pallas_call(
        body,
        grid=(n // bn,),
        in_specs=[
            pl.BlockSpec((bn, nh), lambda i: (i, 0)),
            pl.BlockSpec((bn, nh), lambda i: (i + nb, 0)),
            pl.BlockSpec((bn, dc), lambda i: (i, 0)),
            pl.BlockSpec((bn, dc), lambda i: (i + nb, 0)),
            pl.BlockSpec((1, nh), lambda i: (0, 0)),
        ],
        out_specs=[
            pl.BlockSpec((bn, nh), lambda i: (i, 0)),
            pl.BlockSpec((bn, nhid), lambda i: (i, 0)),
            pl.BlockSpec((bn, nhid), lambda i: (i, 0)),
        ],
        out_shape=[
            jax.ShapeDtypeStruct((n, nh), jnp.float32),
            jax.ShapeDtypeStruct((n, nhid), jnp.float32),
            jax.ShapeDtypeStruct((n, nhid), jnp.float32),
        ],
    )(T, T, CD, CD, b2)


def _tc_finalize(S, Xc, Xsq, np_, bn):
    n, nhid = Xc.shape
    nh = S.shape[1]
    nb = np_ // bn

    def body(s0_ref, s1_ref, xc_ref, xsq_ref, o_ref):
        sv = s0_ref[...] + s1_ref[...]
        rid = lax.broadcasted_iota(jnp.int32, (nh, nhid), 0)
        cid = lax.broadcasted_iota(jnp.int32, (nh, nhid), 1)
        p_a = jnp.where(rid == cid, 1.0, 0.0).astype(jnp.float32)
        p_b = jnp.where(rid == cid + nhid, 1.0, 0.0).astype(jnp.float32)
        rid1 = lax.broadcasted_iota(jnp.int32, (nh, 1), 0)
        p_c = jnp.where((rid1 >= 2 * nhid) & (rid1 < 2 * nhid + _L),
                        1.0 / _L, 0.0).astype(jnp.float32)
        asum = jnp.dot(sv, p_a, preferred_element_type=jnp.float32)
        bsum = jnp.dot(sv, p_b, preferred_element_type=jnp.float32)
        craw = jnp.dot(sv, p_c, preferred_element_type=jnp.float32)
        c1v = jnp.maximum(craw, 1.0)
        s = craw * xsq_ref[...] - 2.0 * xc_ref[...] * asum + bsum
        o_ref[...] = jnp.tanh(s / c1v)

    return pl.pallas_call(
        body,
        grid=(n // bn,),
        in_specs=[
            pl.BlockSpec((bn, nh), lambda i: (i, 0)),
            pl.BlockSpec((bn, nh), lambda i: (i + nb, 0)),
            pl.BlockSpec((bn, nhid), lambda i: (i, 0)),
            pl.BlockSpec((bn, nhid), lambda i: (i, 0)),
        ],
        out_specs=pl.BlockSpec((bn, nhid), lambda i: (i, 0)),
        out_shape=jax.ShapeDtypeStruct((n, nhid), jnp.float32),
    )(S, S, Xc, Xsq)


def kernel(X, edge_index, W, b):
    n, _ = X.shape
    e = edge_index.shape[1]
    nh = W.shape[1]
    nhid = nh // _NHEADS
    bn, np_ = _pick_blocking(n, W.shape[1])

    src = edge_index[0]
    dst = edge_index[1]

    cd = _make_counts(n, np_, e, nh)(dst)
    hn = _tc_scale_matmul(X, W, cd, np_, bn)
    rs = _make_rowsum(n, np_, e, nh)
    T = rs(hn, src, dst)
    Y2, Xc, Xsq = _tc_elu_headmean(n, T, cd, b.reshape(1, nh), np_, bn)
    S = rs(Y2, dst, src)
    return _tc_finalize(S, Xc, Xsq, np_, bn)


# speedup vs baseline: 14.1462x; 1.0002x over previous
import functools
import math

import jax
import jax.numpy as jnp
from jax import lax
from jax.experimental import pallas as pl
from jax.experimental.pallas import tpu as pltpu
from jax.experimental.pallas import tpu_sc as plsc

_NC = 2
_NS = 16
_L = 16
_NW = _NC * _NS
_NHEADS = 4


def _pick_blocking(n: int, d: int) -> tuple[int, int]:
    budget_words = 1_600_000
    best = None
    for bn in range(512, 7, -8):
        if n % bn:
            continue
        q = math.lcm(_NS * 8, bn)
        np_ = ((n + q - 1) // q) * q
        if np_ * d <= budget_words:
            best = (bn, np_)
            break
    if best is None:
        raise ValueError(f"no valid TC blocking for n={n}, d={d}")
    return best


def _chunking(ew: int) -> tuple[int, int]:
    for cs in range(128, 0, -8):
        if ew % cs == 0:
            return cs, ew // cs
    raise ValueError(f"no valid chunking for {ew} edges per worker")


def _stage_chunk(rps: int, cs: int) -> int:
    for q in range(min(rps, cs) // 8 * 8, 0, -8):
        if rps % q == 0:
            return q
    raise ValueError((rps, cs))


def _mesh():
    return plsc.VectorSubcoreMesh(
        core_axis_name="c", subcore_axis_name="s",
        num_cores=_NC, num_subcores=_NS)


def _make_counts(n: int, np_: int, e: int, d: int):
    ew = e // _NW
    assert ew * _NW == e
    cs, nch = _chunking(ew)
    assert nch >= 4
    rps = np_ // _NS
    q = _stage_chunk(rps, cs)

    @functools.partial(
        pl.kernel,
        out_type=jax.ShapeDtypeStruct((_NC * np_, d), jnp.float32),
        mesh=_mesh(),
        scratch_types=[
            pltpu.VMEM((cs,), jnp.int32),
            pltpu.VMEM((cs,), jnp.int32),
            pltpu.VMEM((cs, d), jnp.float32),
            pltpu.VMEM((cs, d), jnp.float32),
            pltpu.VMEM_SHARED((np_, d), jnp.float32),
            pltpu.SemaphoreType.DMA,
            pltpu.SemaphoreType.DMA,
        ],
    )
    def counts(eidx_hbm, out_hbm, ia0, ia1, ones_v, zwbuf, acc, ss0, ss1):
        c = lax.axis_index("c")
        s = lax.axis_index("s")
        wid = s * _NC + c
        sems = (ss0, ss1)
        iabufs = (ia0, ia1)
        base_e = wid * ew

        ones16 = jnp.ones((_L,), jnp.float32)
        zeros16 = jnp.zeros((_L,), jnp.float32)
        for r in range(cs):
            ones_v[r, pl.ds(0, _L)] = ones16
            for k in range(1, d // _L):
                ones_v[r, pl.ds(k * _L, _L)] = zeros16
            for k in range(d // _L):
                zwbuf[r, pl.ds(k * _L, _L)] = zeros16

        base_r = s * rps
        for k in range(rps // q):
            pltpu.sync_copy(zwbuf.at[pl.ds(0, q)],
                            acc.at[pl.ds(base_r + k * q, q)])
        plsc.subcore_barrier()

        def chalf(ci, p, guard):
            def drain():
                pltpu.make_async_copy(
                    ones_v, acc.at[iabufs[p]], sems[p]).wait()

            if guard:
                pl.when(ci >= 2)(drain)
            elif ci >= 2:
                drain()
            pltpu.sync_copy(eidx_hbm.at[pl.ds(base_e + ci * cs, cs)],
                            iabufs[p])
            pltpu.async_copy(ones_v, acc.at[iabufs[p]], sems[p], add=True)

        def pair(g, carry):
            chalf(2 * g, 0, True)
            chalf(2 * g + 1, 1, True)
            return carry

        lax.fori_loop(0, nch // 2, pair, 0)
        if nch % 2:
            chalf(nch - 1, (nch - 1) % 2, False)
        pltpu.make_async_copy(ones_v, acc.at[ia0], sems[0]).wait()
        pltpu.make_async_copy(ones_v, acc.at[ia1], sems[1]).wait()
        plsc.subcore_barrier()

        for k in range(rps // q):
            r0 = base_r + k * q
            pltpu.sync_copy(acc.at[pl.ds(r0, q)], zwbuf.at[pl.ds(0, q)])
            pltpu.sync_copy(zwbuf.at[pl.ds(0, q)],
                            out_hbm.at[pl.ds(c * np_ + r0, q)])

    return counts


def _make_rowsum(n: int, np_: int, e: int, d: int):
    ew = e // _NW
    assert ew * _NW == e
    cs, nch = _chunking(ew)
    rps = np_ // _NS
    q = _stage_chunk(rps, cs)

    assert nch >= 6

    @functools.partial(
        pl.kernel,
        out_type=jax.ShapeDtypeStruct((_NC * np_, d), jnp.float32),
        mesh=_mesh(),
        scratch_types=[
            pltpu.VMEM((cs,), jnp.int32),
            pltpu.VMEM((cs,), jnp.int32),
            pltpu.VMEM((cs,), jnp.int32),
            pltpu.VMEM((cs,), jnp.int32),
            pltpu.VMEM((cs,), jnp.int32),
            pltpu.VMEM((cs,), jnp.int32),
            pltpu.VMEM((cs, d), jnp.float32),
            pltpu.VMEM((cs, d), jnp.float32),
            pltpu.VMEM((cs, d), jnp.float32),
            pltpu.VMEM_SHARED((np_, d), jnp.float32),
            pltpu.SemaphoreType.DMA,
            pltpu.SemaphoreType.DMA,
            pltpu.SemaphoreType.DMA,
            pltpu.SemaphoreType.DMA,
            pltpu.SemaphoreType.DMA,
            pltpu.SemaphoreType.DMA,
            pltpu.SemaphoreType.DMA,
            pltpu.SemaphoreType.DMA,
            pltpu.SemaphoreType.DMA,
        ],
    )
    def rowsum(tab_hbm, gidx_hbm, sidx_hbm, out_hbm,
               ig0, ia0, ig1, ia1, ig2, ia2, rw0, rw1, rw2, acc,
               si0, si1, si2, sg0, sg1, sg2, ss0, ss1, ss2):
        c = lax.axis_index("c")
        s = lax.axis_index("s")
        wid = s * _NC + c
        bufs = ((ig0, ia0, rw0, si0, sg0, ss0),
                (ig1, ia1, rw1, si1, sg1, ss1),
                (ig2, ia2, rw2, si2, sg2, ss2))
        base_e = wid * ew

        zeros16 = jnp.zeros((_L,), jnp.float32)
        for r in range(cs):
            for k in range(d // _L):
                rw0[r, pl.ds(k * _L, _L)] = zeros16

        base_r = s * rps
        for k in range(rps // q):
            pltpu.sync_copy(rw0.at[pl.ds(0, q)],
                            acc.at[pl.ds(base_r + k * q, q)])
        plsc.subcore_barrier()

        def idx_fire(ci, j):
            ig, ia, _, si, _, _ = bufs[j]
            off = base_e + ci * cs
            pltpu.async_copy(gidx_hbm.at[pl.ds(off, cs)], ig, si)
            pltpu.async_copy(sidx_hbm.at[pl.ds(off, cs)], ia, si)

        def idx_wait(ci, j):
            ig, ia, _, si, _, _ = bufs[j]
            off = base_e + ci * cs
            pltpu.make_async_copy(gidx_hbm.at[pl.ds(off, cs)], ig, si).wait()
            pltpu.make_async_copy(sidx_hbm.at[pl.ds(off, cs)], ia, si).wait()

        def gather_fire(j):
            ig, _, rw, _, sg, _ = bufs[j]
            pltpu.async_copy(tab_hbm.at[ig], rw, sg)

        def step(ci, j, guard):
            j1 = (j + 1) % 3
            j2 = (j + 2) % 3
            _, ia, rw, _, sg, ss = bufs[j]
            _, iap, rwp, _, _, ssp = bufs[j2]

            def when(pred, f):
                if guard:
                    pl.when(pred)(f)
                elif pred:
                    f()

            when(ci >= 1, lambda: pltpu.make_async_copy(
                rwp, acc.at[iap], ssp).wait())
            when(ci + 2 < nch, lambda: idx_fire(ci + 2, j2))

            def g1():
                idx_wait(ci + 1, j1)
                gather_fire(j1)

            when(ci + 1 < nch, g1)
            pltpu.make_async_copy(tab_hbm.at[bufs[j][0]], rw, sg).wait()
            pltpu.async_copy(rw, acc.at[ia], ss, add=True)

        idx_fire(0, 0)
        idx_fire(1, 1)
        idx_wait(0, 0)
        gather_fire(0)

        def triple(g, carry):
            ci = 3 * g
            step(ci, 0, True)
            step(ci + 1, 1, True)
            step(ci + 2, 2, True)
            return carry

        lax.fori_loop(0, nch // 3, triple, 0)
        for ci in range(nch // 3 * 3, nch):
            step(ci, ci % 3, False)
        jf = (nch - 1) % 3
        _, iaf, rwf, _, _, ssf = bufs[jf]
        pltpu.make_async_copy(rwf, acc.at[iaf], ssf).wait()
        plsc.subcore_barrier()

        for k in range(rps // q):
            r0 = base_r + k * q
            pltpu.sync_copy(acc.at[pl.ds(r0, q)], rw0.at[pl.ds(0, q)])
            pltpu.sync_copy(rw0.at[pl.ds(0, q)],
                            out_hbm.at[pl.ds(c * np_ + r0, q)])

    return rowsum


def _count_extract(cd, nh):
    rid = lax.broadcasted_iota(jnp.int32, (nh, 1), 0)
    p = jnp.where(rid < _L, 1.0 / _L, 0.0).astype(jnp.float32)
    return jnp.dot(cd, p, preferred_element_type=jnp.float32)


def _tc_scale_matmul(X, W, CD, np_, bn):
    n, dx = X.shape
    nh = W.shape[1]
    nb = np_ // bn

    dc = CD.shape[1]

    def body(x_ref, w_ref, c0_ref, c1_ref, o_ref):
        deg = jnp.maximum(_count_extract(c0_ref[...] + c1_ref[...], dc), 1.0)
        h = jnp.dot(x_ref[...], w_ref[...], preferred_element_type=jnp.float32)
        o_ref[...] = h * lax.rsqrt(deg)

    return pl.pallas_call(
        body,
        grid=(n // bn,),
        in_specs=[
            pl.BlockSpec((bn, dx), lambda i: (i, 0)),
            pl.BlockSpec((dx, nh), lambda i: (0, 0)),
            pl.BlockSpec((bn, dc), lambda i: (i, 0)),
            pl.BlockSpec((bn, dc), lambda i: (i + nb, 0)),
        ],
        out_specs=pl.BlockSpec((bn, nh), lambda i: (i, 0)),
        out_shape=jax.ShapeDtypeStruct((n, nh), jnp.float32),
    )(X, W, CD, CD)


def _tc_elu_headmean(n, T, CD, b2, np_, bn):
    nh = T.shape[1]
    nhid = nh // _NHEADS
    nb = np_ // bn

    dc = CD.shape[1]

    def body(t0_ref, t1_ref, c0_ref, c1_ref, b_ref, y2_ref, xc_ref, xsq_ref):
        deg = jnp.maximum(_count_extract(c0_ref[...] + c1_ref[...], dc), 1.0)
        agg = (t0_ref[...] + t1_ref[...]) * lax.rsqrt(deg) + b_ref[...]
        el = jnp.where(agg > 0.0, agg, jnp.exp(jnp.minimum(agg, 0.0)) - 1.0)
        rid = lax.broadcasted_iota(jnp.int32, (nh, nhid), 0) // _NHEADS
        cid = lax.broadcasted_iota(jnp.int32, (nh, nhid), 1)
        m = jnp.where(rid == cid, 1.0 / _NHEADS, 0.0).astype(jnp.float32)
        xc = jnp.dot(el, m, preferred_element_type=jnp.float32)
        xsq = xc * xc
        ones_blk = jnp.ones((xc.shape[0], _L), jnp.float32)
        pad = jnp.zeros((xc.shape[0], nh - 2 * nhid - _L), jnp.float32)
        y2_ref[...] = jnp.concatenate([xc, xsq, ones_blk, pad], axis=1)
        xc_ref[...] = xc
        xsq_ref[...] = xsq

    return pl.pallas_call(
        body,
        grid=(n // bn,),
        in_specs=[
            pl.BlockSpec((bn, nh), lambda i: (i, 0)),
            pl.BlockSpec((bn, nh), lambda i: (i + nb, 0)),
            pl.BlockSpec((bn, dc), lambda i: (i, 0)),
            pl.BlockSpec((bn, dc), lambda i: (i + nb, 0)),
            pl.BlockSpec((1, nh), lambda i: (0, 0)),
        ],
        out_specs=[
            pl.BlockSpec((bn, nh), lambda i: (i, 0)),
            pl.BlockSpec((bn, nhid), lambda i: (i, 0)),
            pl.BlockSpec((bn, nhid), lambda i: (i, 0)),
        ],
        out_shape=[
            jax.ShapeDtypeStruct((n, nh), jnp.float32),
            jax.ShapeDtypeStruct((n, nhid), jnp.float32),
            jax.ShapeDtypeStruct((n, nhid), jnp.float32),
        ],
    )(T, T, CD, CD, b2)


def _tc_finalize(S, Xc, Xsq, np_, bn):
    n, nhid = Xc.shape
    nh = S.shape[1]
    nb = np_ // bn

    def body(s0_ref, s1_ref, xc_ref, xsq_ref, o_ref):
        sv = s0_ref[...] + s1_ref[...]
        rid = lax.broadcasted_iota(jnp.int32, (nh, nhid), 0)
        cid = lax.broadcasted_iota(jnp.int32, (nh, nhid), 1)
        p_a = jnp.where(rid == cid, 1.0, 0.0).astype(jnp.float32)
        p_b = jnp.where(rid == cid + nhid, 1.0, 0.0).astype(jnp.float32)
        rid1 = lax.broadcasted_iota(jnp.int32, (nh, 1), 0)
        p_c = jnp.where((rid1 >= 2 * nhid) & (rid1 < 2 * nhid + _L),
                        1.0 / _L, 0.0).astype(jnp.float32)
        asum = jnp.dot(sv, p_a, preferred_element_type=jnp.float32)
        bsum = jnp.dot(sv, p_b, preferred_element_type=jnp.float32)
        craw = jnp.dot(sv, p_c, preferred_element_type=jnp.float32)
        c1v = jnp.maximum(craw, 1.0)
        s = craw * xsq_ref[...] - 2.0 * xc_ref[...] * asum + bsum
        o_ref[...] = jnp.tanh(s / c1v)

    return pl.pallas_call(
        body,
        grid=(n // bn,),
        in_specs=[
            pl.BlockSpec((bn, nh), lambda i: (i, 0)),
            pl.BlockSpec((bn, nh), lambda i: (i + nb, 0)),
            pl.BlockSpec((bn, nhid), lambda i: (i, 0)),
            pl.BlockSpec((bn, nhid), lambda i: (i, 0)),
        ],
        out_specs=pl.BlockSpec((bn, nhid), lambda i: (i, 0)),
        out_shape=jax.ShapeDtypeStruct((n, nhid), jnp.float32),
    )(S, S, Xc, Xsq)


def kernel(X, edge_index, W, b):
    n, _ = X.shape
    e = edge_index.shape[1]
    nh = W.shape[1]
    nhid = nh // _NHEADS
    bn, np_ = _pick_blocking(n, W.shape[1])

    src = edge_index[0]
    dst = edge_index[1]

    cd = _make_counts(n, np_, e, nh)(dst)
    hn = _tc_scale_matmul(X, W, cd, np_, bn)
    rs = _make_rowsum(n, np_, e, nh)
    T = rs(hn, src, dst)
    Y2, Xc, Xsq = _tc_elu_headmean(n, T, cd, b.reshape(1, nh), np_, bn)
    S = rs(Y2, dst, src)
    return _tc_finalize(S, Xc, Xsq, np_, bn)
